# baseline (speedup 1.0000x reference)
import functools

import jax
import jax.numpy as jnp
from jax import lax
from jax.experimental import pallas as pl
from jax.experimental.pallas import tpu as pltpu
from jax.experimental.pallas import tpu_sc as plsc

N_NODES = 10000
N_PAD = 10240
N_EDGES = 320000
E_PAD = 327680
IN_DIM = 128
HIDDEN = 256
OUT_DIM = 12
N_GRAPHS = 64

NC = 2
NS = 16
CHUNK = 128
DEG_CHUNKS = E_PAD // (NC * NS * CHUNK)
EDGE_CHUNKS = E_PAD // (NS * CHUNK)
IBLK = 32
N_IBLK = EDGE_CHUNKS // IBLK
ROWS_PER_TILE = N_PAD // NS
RBLK = 2048
N_RBLK = N_PAD // RBLK

_mesh = plsc.VectorSubcoreMesh(core_axis_name="c", subcore_axis_name="s")


@functools.partial(
    pl.kernel,
    out_type=jax.ShapeDtypeStruct((NC * N_PAD,), jnp.float32),
    mesh=_mesh,
    scratch_types=[
        pltpu.VMEM((DEG_CHUNKS * CHUNK,), jnp.int32),
        pltpu.VMEM((DEG_CHUNKS, CHUNK), jnp.int32),
        pltpu.VMEM((ROWS_PER_TILE,), jnp.float32),
        pltpu.VMEM((CHUNK,), jnp.float32),
        pltpu.VMEM_SHARED((N_PAD,), jnp.float32),
    ],
)
def _sc_degree(dst_hbm, out_hbm, idxf_v, idx_v, z_v, ones_v, hist_sh):
    c = lax.axis_index("c")
    s = lax.axis_index("s")
    wid = c * NS + s
    pltpu.sync_copy(
        dst_hbm.at[pl.ds(wid * DEG_CHUNKS * CHUNK, DEG_CHUNKS * CHUNK)],
        idxf_v)

    def _repack(j, carry):
        for k in range(CHUNK // 16):
            idx_v[j, pl.ds(k * 16, 16)] = idxf_v[pl.ds(j * CHUNK + k * 16, 16)]
        return carry
    lax.fori_loop(0, DEG_CHUNKS, _repack, 0)

    def _fill_z(i, carry):
        z_v[pl.ds(i * 16, 16)] = jnp.zeros((16,), jnp.float32)
        return carry
    lax.fori_loop(0, ROWS_PER_TILE // 16, _fill_z, 0)
    for k in range(CHUNK // 16):
        ones_v[pl.ds(k * 16, 16)] = jnp.ones((16,), jnp.float32)

    pltpu.sync_copy(z_v, hist_sh.at[pl.ds(s * ROWS_PER_TILE, ROWS_PER_TILE)])
    plsc.subcore_barrier()

    def _hist(j, carry):
        pltpu.sync_copy(ones_v, hist_sh.at[idx_v.at[j]], add=True)
        return carry
    lax.fori_loop(0, DEG_CHUNKS, _hist, 0)
    plsc.subcore_barrier()

    pltpu.sync_copy(
        hist_sh.at[pl.ds(s * ROWS_PER_TILE, ROWS_PER_TILE)],
        out_hbm.at[pl.ds(c * N_PAD + s * ROWS_PER_TILE, ROWS_PER_TILE)],
    )


@functools.partial(
    pl.kernel,
    out_type=jax.ShapeDtypeStruct((NC * N_PAD, IN_DIM), jnp.float32),
    mesh=_mesh,
    scratch_types=[
        pltpu.VMEM((IBLK * CHUNK,), jnp.int32),
        pltpu.VMEM((IBLK * CHUNK,), jnp.int32),
        pltpu.VMEM((IBLK, CHUNK), jnp.int32),
        pltpu.VMEM((CHUNK, IN_DIM), jnp.float32),
        pltpu.VMEM((CHUNK, IN_DIM), jnp.float32),
        pltpu.VMEM_SHARED((N_PAD, IN_DIM), jnp.float32),
        pltpu.SemaphoreType.DMA,
        pltpu.SemaphoreType.DMA,
    ],
)
def _sc_edge(g_hbm, src_hbm, dst_hbm, out_hbm, si_v, dif_v, di_v, buf_a,
             buf_b, acc_sh, sem_ga, sem_gb):
    c = lax.axis_index("c")
    s = lax.axis_index("s")

    base = c * N_PAD + s * ROWS_PER_TILE
    pltpu.sync_copy(
        g_hbm.at[pl.ds(base, ROWS_PER_TILE)],
        acc_sh.at[pl.ds(s * ROWS_PER_TILE, ROWS_PER_TILE)],
    )
    plsc.subcore_barrier()

    def _gref(j):
        return g_hbm.at[si_v.at[pl.ds(j * CHUNK, CHUNK)]]

    def _block(j0, carry):
        off = (s * EDGE_CHUNKS + j0 * IBLK) * CHUNK
        pltpu.sync_copy(src_hbm.at[pl.ds(off, IBLK * CHUNK)], si_v)
        pltpu.sync_copy(dst_hbm.at[pl.ds(off, IBLK * CHUNK)], dif_v)

        @pl.when(c == 1)
        def _():
            def _shift(j, cc):
                sl = pl.ds(j * 16, 16)
                si_v[sl] = si_v[sl] + N_PAD
                return cc
            lax.fori_loop(0, IBLK * CHUNK // 16, _shift, 0)

        def _repack(j, cc):
            for k in range(CHUNK // 16):
                di_v[j, pl.ds(k * 16, 16)] = dif_v[pl.ds(j * CHUNK + k * 16, 16)]
            return cc
        lax.fori_loop(0, IBLK, _repack, 0)

        pltpu.async_copy(_gref(0), buf_a, sem_ga)

        def _pair(j, cc):
            a = 2 * j
            b = a + 1
            pltpu.async_copy(_gref(b), buf_b, sem_gb)
            pltpu.make_async_copy(_gref(a), buf_a, sem_ga).wait()
            pltpu.sync_copy(buf_a, acc_sh.at[di_v.at[a]], add=True)

            @pl.when(j < IBLK // 2 - 1)
            def _():
                pltpu.async_copy(_gref(a + 2), buf_a, sem_ga)

            pltpu.make_async_copy(_gref(b), buf_b, sem_gb).wait()
            pltpu.sync_copy(buf_b, acc_sh.at[di_v.at[b]], add=True)
            return cc
        lax.fori_loop(0, IBLK // 2, _pair, 0)
        return carry
    lax.fori_loop(0, N_IBLK, _block, 0)
    plsc.subcore_barrier()

    pltpu.sync_copy(
        acc_sh.at[pl.ds(s * ROWS_PER_TILE, ROWS_PER_TILE)],
        out_hbm.at[pl.ds(base, ROWS_PER_TILE)],
    )


def _tc1_body(x_ref, w_ref, deg_ref, g_ref, dinv_ref):
    i = pl.program_id(0)
    degsum = deg_ref[0, 0] + deg_ref[1, 0] + 1.0
    rows = lax.broadcasted_iota(jnp.int32, (RBLK, 1), 0) + i * RBLK
    dinv = jnp.where(rows < N_NODES, lax.rsqrt(degsum), 0.0)
    dinv_ref[0] = dinv
    xw = jnp.dot(x_ref[...], w_ref[...], preferred_element_type=jnp.float32)
    g = xw * dinv
    g_ref[0, 0] = g[:, :IN_DIM]
    g_ref[1, 0] = g[:, IN_DIM:]


def _tc_layer1(xp, W1, deg4):
    return pl.pallas_call(
        _tc1_body,
        grid=(N_RBLK,),
        in_specs=[
            pl.BlockSpec((RBLK, IN_DIM), lambda i: (i, 0)),
            pl.BlockSpec((IN_DIM, HIDDEN), lambda i: (0, 0)),
            pl.BlockSpec((2, 1, RBLK, 1), lambda i: (0, i, 0, 0)),
        ],
        out_specs=[
            pl.BlockSpec((2, 1, RBLK, IN_DIM), lambda i: (0, i, 0, 0)),
            pl.BlockSpec((1, RBLK, 1), lambda i: (i, 0, 0)),
        ],
        out_shape=[
            jax.ShapeDtypeStruct((2, N_RBLK, RBLK, IN_DIM), jnp.float32),
            jax.ShapeDtypeStruct((N_RBLK, RBLK, 1), jnp.float32),
        ],
    )(xp, W1, deg4)


def _tc2_body(acc_ref, dv_ref, b1_ref, w_ref, g_ref):
    dinv = dv_ref[0]
    h = jnp.concatenate([acc_ref[0, 0], acc_ref[1, 0]], axis=1)
    h1 = jax.nn.relu(h * dinv + b1_ref[...])
    hw = jnp.dot(h1, w_ref[...], preferred_element_type=jnp.float32)
    g = hw * dinv
    g_ref[0, 0] = g[:, :IN_DIM]
    g_ref[1, 0] = g[:, IN_DIM:]


def _tc_layer2(acc4, dinv3, b1r, W2):
    return pl.pallas_call(
        _tc2_body,
        grid=(N_RBLK,),
        in_specs=[
            pl.BlockSpec((2, 1, RBLK, IN_DIM), lambda i: (0, i, 0, 0)),
            pl.BlockSpec((1, RBLK, 1), lambda i: (i, 0, 0)),
            pl.BlockSpec((1, HIDDEN), lambda i: (0, 0)),
            pl.BlockSpec((HIDDEN, HIDDEN), lambda i: (0, 0)),
        ],
        out_specs=pl.BlockSpec((2, 1, RBLK, IN_DIM), lambda i: (0, i, 0, 0)),
        out_shape=jax.ShapeDtypeStruct((2, N_RBLK, RBLK, IN_DIM), jnp.float32),
    )(acc4, dinv3, b1r, W2)


def _tc3_body(acc_ref, dv_ref, b2_ref, bt_ref, w3_ref, b3_ref, out_ref, pooled):
    i = pl.program_id(0)
    dinv = dv_ref[0]
    h = jnp.concatenate([acc_ref[0, 0], acc_ref[1, 0]], axis=1)
    h2 = jax.nn.relu(h * dinv + b2_ref[...])
    seg = lax.broadcasted_iota(jnp.int32, (RBLK, N_GRAPHS), 1)
    oh = (bt_ref[0] == seg).astype(jnp.float32)
    part = lax.dot_general(oh, h2, (((0,), (0,)), ((), ())),
                           preferred_element_type=jnp.float32)

    @pl.when(i == 0)
    def _():
        pooled[...] = jnp.zeros_like(pooled)

    pooled[...] += part

    @pl.when(i == N_RBLK - 1)
    def _():
        out_ref[...] = (
            jnp.dot(pooled[...], w3_ref[...], preferred_element_type=jnp.float32)
            + b3_ref[...]
        )


def _tc_head(acc4, dinv3, b2r, batchr, W3, b3r):
    return pl.pallas_call(
        _tc3_body,
        grid=(N_RBLK,),
        in_specs=[
            pl.BlockSpec((2, 1, RBLK, IN_DIM), lambda i: (0, i, 0, 0)),
            pl.BlockSpec((1, RBLK, 1), lambda i: (i, 0, 0)),
            pl.BlockSpec((1, HIDDEN), lambda i: (0, 0)),
            pl.BlockSpec((1, RBLK, 1), lambda i: (i, 0, 0)),
            pl.BlockSpec((HIDDEN, OUT_DIM), lambda i: (0, 0)),
            pl.BlockSpec((1, OUT_DIM), lambda i: (0, 0)),
        ],
        out_specs=pl.BlockSpec((N_GRAPHS, OUT_DIM), lambda i: (0, 0)),
        out_shape=jax.ShapeDtypeStruct((N_GRAPHS, OUT_DIM), jnp.float32),
        scratch_shapes=[pltpu.VMEM((N_GRAPHS, HIDDEN), jnp.float32)],
    )(acc4, dinv3, b2r, batchr, W3, b3r)


def kernel(x, edge_index, batch, W1, b1, W2, b2, W3, b3):
    ei = edge_index.astype(jnp.int32)
    n_extra = E_PAD - N_EDGES
    pad_idx = N_NODES + (jnp.arange(n_extra, dtype=jnp.int32) % (N_PAD - N_NODES))
    src_e = jnp.concatenate([ei[0], pad_idx])
    dst_e = jnp.concatenate([ei[1], pad_idx])

    xp = jnp.pad(x, ((0, N_PAD - N_NODES), (0, 0)))
    batch_p = jnp.pad(batch.astype(jnp.int32), (0, N_PAD - N_NODES),
                      constant_values=N_GRAPHS).reshape(N_RBLK, RBLK, 1)
    b1r = b1.reshape(1, HIDDEN)
    b2r = b2.reshape(1, HIDDEN)
    b3r = b3.reshape(1, OUT_DIM)

    deg = _sc_degree(dst_e)
    deg4 = deg.reshape(NC, N_RBLK, RBLK, 1)
    g1, dinv3 = _tc_layer1(xp, W1, deg4)
    acc1 = _sc_edge(g1.reshape(NC * N_PAD, IN_DIM), src_e, dst_e)
    g2 = _tc_layer2(acc1.reshape(NC, N_RBLK, RBLK, IN_DIM), dinv3, b1r, W2)
    acc2 = _sc_edge(g2.reshape(NC * N_PAD, IN_DIM), src_e, dst_e)
    return _tc_head(acc2.reshape(NC, N_RBLK, RBLK, IN_DIM), dinv3, b2r,
                    batch_p, W3, b3r)


# speedup vs baseline: 1.0590x; 1.0590x over previous
import functools

import jax
import jax.numpy as jnp
from jax import lax
from jax.experimental import pallas as pl
from jax.experimental.pallas import tpu as pltpu
from jax.experimental.pallas import tpu_sc as plsc

N_NODES = 10000
N_PAD = 10240
N_EDGES = 320000
E_PAD = 327680
IN_DIM = 128
HIDDEN = 256
OUT_DIM = 12
N_GRAPHS = 64

NC = 2
NS = 16
CHUNK = 128
DEG_CHUNKS = E_PAD // (NC * NS * CHUNK)
EDGE_CHUNKS = E_PAD // (NS * CHUNK)
IBLK = 40
N_IBLK = EDGE_CHUNKS // IBLK
ROWS_PER_TILE = N_PAD // NS
RBLK = 2048
N_RBLK = N_PAD // RBLK

_mesh = plsc.VectorSubcoreMesh(core_axis_name="c", subcore_axis_name="s")


@functools.partial(
    pl.kernel,
    out_type=jax.ShapeDtypeStruct((NC * N_PAD,), jnp.float32),
    mesh=_mesh,
    scratch_types=[
        pltpu.VMEM((DEG_CHUNKS, CHUNK), jnp.int32),
        pltpu.VMEM((ROWS_PER_TILE,), jnp.float32),
        pltpu.VMEM((CHUNK,), jnp.float32),
        pltpu.VMEM_SHARED((N_PAD,), jnp.float32),
    ],
)
def _sc_degree(dst_hbm, out_hbm, idx_v, z_v, ones_v, hist_sh):
    c = lax.axis_index("c")
    s = lax.axis_index("s")
    wid = c * NS + s
    pltpu.sync_copy(
        dst_hbm.at[wid // 2, pl.ds((wid % 2) * DEG_CHUNKS, DEG_CHUNKS)], idx_v)

    def _fill_z(i, carry):
        z_v[pl.ds(i * 16, 16)] = jnp.zeros((16,), jnp.float32)
        return carry
    lax.fori_loop(0, ROWS_PER_TILE // 16, _fill_z, 0)
    for k in range(CHUNK // 16):
        ones_v[pl.ds(k * 16, 16)] = jnp.ones((16,), jnp.float32)

    pltpu.sync_copy(z_v, hist_sh.at[pl.ds(s * ROWS_PER_TILE, ROWS_PER_TILE)])
    plsc.subcore_barrier()

    def _hist(j, carry):
        pltpu.sync_copy(ones_v, hist_sh.at[idx_v.at[j]], add=True)
        return carry
    lax.fori_loop(0, DEG_CHUNKS, _hist, 0)
    plsc.subcore_barrier()

    pltpu.sync_copy(
        hist_sh.at[pl.ds(s * ROWS_PER_TILE, ROWS_PER_TILE)],
        out_hbm.at[pl.ds(c * N_PAD + s * ROWS_PER_TILE, ROWS_PER_TILE)],
    )


@functools.partial(
    pl.kernel,
    out_type=jax.ShapeDtypeStruct((NC * N_PAD, IN_DIM), jnp.float32),
    mesh=_mesh,
    scratch_types=[
        pltpu.VMEM((IBLK, CHUNK), jnp.int32),
        pltpu.VMEM((IBLK, CHUNK), jnp.int32),
        pltpu.VMEM((CHUNK, IN_DIM), jnp.float32),
        pltpu.VMEM((CHUNK, IN_DIM), jnp.float32),
        pltpu.VMEM_SHARED((N_PAD, IN_DIM), jnp.float32),
        pltpu.SemaphoreType.DMA,
        pltpu.SemaphoreType.DMA,
    ],
)
def _sc_edge(g_hbm, src_hbm, dst_hbm, out_hbm, si_v, di_v, buf_a, buf_b,
             acc_sh, sem_ga, sem_gb):
    c = lax.axis_index("c")
    s = lax.axis_index("s")

    base = c * N_PAD + s * ROWS_PER_TILE
    pltpu.sync_copy(
        g_hbm.at[pl.ds(base, ROWS_PER_TILE)],
        acc_sh.at[pl.ds(s * ROWS_PER_TILE, ROWS_PER_TILE)],
    )
    plsc.subcore_barrier()

    def _block(j0, carry):
        pltpu.sync_copy(src_hbm.at[s, pl.ds(j0 * IBLK, IBLK)], si_v)
        pltpu.sync_copy(dst_hbm.at[s, pl.ds(j0 * IBLK, IBLK)], di_v)

        @pl.when(c == 1)
        def _():
            def _shift(j, cc):
                for k in range(CHUNK // 16):
                    sl = pl.ds(k * 16, 16)
                    si_v[j, sl] = si_v[j, sl] + N_PAD
                return cc
            lax.fori_loop(0, IBLK, _shift, 0)

        pltpu.async_copy(g_hbm.at[si_v.at[0]], buf_a, sem_ga)

        def _pair(j, cc):
            a = 2 * j
            b = a + 1
            pltpu.async_copy(g_hbm.at[si_v.at[b]], buf_b, sem_gb)
            pltpu.make_async_copy(g_hbm.at[si_v.at[a]], buf_a, sem_ga).wait()
            pltpu.sync_copy(buf_a, acc_sh.at[di_v.at[a]], add=True)

            @pl.when(j < IBLK // 2 - 1)
            def _():
                pltpu.async_copy(g_hbm.at[si_v.at[a + 2]], buf_a, sem_ga)

            pltpu.make_async_copy(g_hbm.at[si_v.at[b]], buf_b, sem_gb).wait()
            pltpu.sync_copy(buf_b, acc_sh.at[di_v.at[b]], add=True)
            return cc
        lax.fori_loop(0, IBLK // 2, _pair, 0)
        return carry
    lax.fori_loop(0, N_IBLK, _block, 0)
    plsc.subcore_barrier()

    pltpu.sync_copy(
        acc_sh.at[pl.ds(s * ROWS_PER_TILE, ROWS_PER_TILE)],
        out_hbm.at[pl.ds(base, ROWS_PER_TILE)],
    )


def _to_col(v16):
    e1 = (lax.broadcasted_iota(jnp.int32, (RBLK, 16), 0) // 128
          == lax.broadcasted_iota(jnp.int32, (RBLK, 16), 1)).astype(jnp.float32)
    a = jnp.dot(e1, v16, preferred_element_type=jnp.float32)
    m = (lax.broadcasted_iota(jnp.int32, (RBLK, 128), 0) % 128
         == lax.broadcasted_iota(jnp.int32, (RBLK, 128), 1)).astype(jnp.float32)
    return jnp.dot(a * m, jnp.ones((128, 1), jnp.float32),
                   preferred_element_type=jnp.float32)


def _dinv_block(deg_ref, i):
    d = _to_col(deg_ref[0, 0] + deg_ref[1, 0] + 1.0)
    r = lax.broadcasted_iota(jnp.int32, (RBLK, 1), 0) + i * RBLK
    return jnp.where(r < N_NODES, lax.rsqrt(d), 0.0)


def _tc1_body(x_ref, w_ref, deg_ref, g_ref):
    dinv = _dinv_block(deg_ref, pl.program_id(0))
    xw = jnp.dot(x_ref[...], w_ref[...], preferred_element_type=jnp.float32)
    g = xw * dinv
    g_ref[0, 0] = g[:, :IN_DIM]
    g_ref[1, 0] = g[:, IN_DIM:]


def _tc_layer1(xp, W1, deg4):
    return pl.pallas_call(
        _tc1_body,
        grid=(N_RBLK,),
        in_specs=[
            pl.BlockSpec((RBLK, IN_DIM), lambda i: (i, 0)),
            pl.BlockSpec((IN_DIM, HIDDEN), lambda i: (0, 0)),
            pl.BlockSpec((2, 1, 16, 128), lambda i: (0, i, 0, 0)),
        ],
        out_specs=pl.BlockSpec((2, 1, RBLK, IN_DIM), lambda i: (0, i, 0, 0)),
        out_shape=jax.ShapeDtypeStruct((2, N_RBLK, RBLK, IN_DIM), jnp.float32),
    )(xp, W1, deg4)


def _tc2_body(acc_ref, deg_ref, b1_ref, w_ref, g_ref):
    dinv = _dinv_block(deg_ref, pl.program_id(0))
    h = jnp.concatenate([acc_ref[0, 0], acc_ref[1, 0]], axis=1)
    h1 = jax.nn.relu(h * dinv + b1_ref[...])
    hw = jnp.dot(h1, w_ref[...], preferred_element_type=jnp.float32)
    g = hw * dinv
    g_ref[0, 0] = g[:, :IN_DIM]
    g_ref[1, 0] = g[:, IN_DIM:]


def _tc_layer2(acc4, deg4, b1r, W2):
    return pl.pallas_call(
        _tc2_body,
        grid=(N_RBLK,),
        in_specs=[
            pl.BlockSpec((2, 1, RBLK, IN_DIM), lambda i: (0, i, 0, 0)),
            pl.BlockSpec((2, 1, 16, 128), lambda i: (0, i, 0, 0)),
            pl.BlockSpec((1, HIDDEN), lambda i: (0, 0)),
            pl.BlockSpec((HIDDEN, HIDDEN), lambda i: (0, 0)),
        ],
        out_specs=pl.BlockSpec((2, 1, RBLK, IN_DIM), lambda i: (0, i, 0, 0)),
        out_shape=jax.ShapeDtypeStruct((2, N_RBLK, RBLK, IN_DIM), jnp.float32),
    )(acc4, deg4, b1r, W2)


def _tc3_body(acc_ref, deg_ref, b2_ref, bt_ref, w3_ref, b3_ref, out_ref,
              pooled):
    i = pl.program_id(0)
    dinv = _dinv_block(deg_ref, i)
    h = jnp.concatenate([acc_ref[0, 0], acc_ref[1, 0]], axis=1)
    h2 = jax.nn.relu(h * dinv + b2_ref[...])
    bt = _to_col(bt_ref[0].astype(jnp.float32))
    seg = lax.broadcasted_iota(jnp.int32, (RBLK, N_GRAPHS), 1).astype(jnp.float32)
    oh = (bt == seg).astype(jnp.float32)
    part = lax.dot_general(oh, h2, (((0,), (0,)), ((), ())),
                           preferred_element_type=jnp.float32)

    @pl.when(i == 0)
    def _():
        pooled[...] = jnp.zeros_like(pooled)

    pooled[...] += part

    @pl.when(i == N_RBLK - 1)
    def _():
        out_ref[...] = (
            jnp.dot(pooled[...], w3_ref[...], preferred_element_type=jnp.float32)
            + b3_ref[...]
        )


def _tc_head(acc4, deg4, b2r, batchr, W3, b3r):
    return pl.pallas_call(
        _tc3_body,
        grid=(N_RBLK,),
        in_specs=[
            pl.BlockSpec((2, 1, RBLK, IN_DIM), lambda i: (0, i, 0, 0)),
            pl.BlockSpec((2, 1, 16, 128), lambda i: (0, i, 0, 0)),
            pl.BlockSpec((1, HIDDEN), lambda i: (0, 0)),
            pl.BlockSpec((1, 16, 128), lambda i: (i, 0, 0)),
            pl.BlockSpec((HIDDEN, OUT_DIM), lambda i: (0, 0)),
            pl.BlockSpec((1, OUT_DIM), lambda i: (0, 0)),
        ],
        out_specs=pl.BlockSpec((N_GRAPHS, OUT_DIM), lambda i: (0, 0)),
        out_shape=jax.ShapeDtypeStruct((N_GRAPHS, OUT_DIM), jnp.float32),
        scratch_shapes=[pltpu.VMEM((N_GRAPHS, HIDDEN), jnp.float32)],
    )(acc4, deg4, b2r, batchr, W3, b3r)


def kernel(x, edge_index, batch, W1, b1, W2, b2, W3, b3):
    ei = edge_index.astype(jnp.int32)
    n_extra = E_PAD - N_EDGES
    pad_idx = N_NODES + (jnp.arange(n_extra, dtype=jnp.int32) % (N_PAD - N_NODES))
    src_e = jnp.concatenate([ei[0], pad_idx]).reshape(NS, EDGE_CHUNKS, CHUNK)
    dst_e = jnp.concatenate([ei[1], pad_idx]).reshape(NS, EDGE_CHUNKS, CHUNK)

    xp = jnp.pad(x, ((0, N_PAD - N_NODES), (0, 0)))
    batch_p = jnp.pad(batch.astype(jnp.int32), (0, N_PAD - N_NODES),
                      constant_values=N_GRAPHS).reshape(N_RBLK, 16, 128)
    b1r = b1.reshape(1, HIDDEN)
    b2r = b2.reshape(1, HIDDEN)
    b3r = b3.reshape(1, OUT_DIM)

    deg = _sc_degree(dst_e)
    deg4 = deg.reshape(NC, N_RBLK, 16, 128)
    g1 = _tc_layer1(xp, W1, deg4)
    acc1 = _sc_edge(g1.reshape(NC * N_PAD, IN_DIM), src_e, dst_e)
    g2 = _tc_layer2(acc1.reshape(NC, N_RBLK, RBLK, IN_DIM), deg4, b1r, W2)
    acc2 = _sc_edge(g2.reshape(NC * N_PAD, IN_DIM), src_e, dst_e)
    return _tc_head(acc2.reshape(NC, N_RBLK, RBLK, IN_DIM), deg4, b2r,
                    batch_p, W3, b3r)
